# Initial kernel scaffold; baseline (speedup 1.0000x reference)
import jax
import jax.numpy as jnp
from jax.experimental import pallas as pl


def kernel(boxes, classes):
    raise NotImplementedError("write your pallas kernel here")


# speedup vs baseline: 2.0340x; 2.0340x over previous
import functools

import jax
import jax.numpy as jnp
from jax.experimental import pallas as pl
from jax.experimental.pallas import tpu as pltpu

_N = 20000
_C = 80
_MAX_DET = 300
_IOU_THR = 0.5
_SCORE_THR = 0.05
_LANE = 128
_NP = ((_N + _LANE - 1) // _LANE) * _LANE
_RP = 304

_BIG_I = 2**30


def _nms_body(rows_ref, scores_ref, out_s_ref, out_l_ref, out_b_ref,
              s_ref, rec_s_ref, rec_i_ref):
    x1 = rows_ref[0:1, :]
    y1 = rows_ref[1:2, :]
    x2 = rows_ref[2:3, :]
    y2 = rows_ref[3:4, :]
    areas = jnp.maximum(x2 - x1, 0.0) * jnp.maximum(y2 - y1, 0.0)

    raw = scores_ref[...]
    s_ref[...] = jnp.where(raw > _SCORE_THR, raw, -1.0)
    rec_s_ref[...] = jnp.full((_C, _RP), -1.0, jnp.float32)
    rec_i_ref[...] = jnp.full((_C, _RP), -1, jnp.int32)

    col = jax.lax.broadcasted_iota(jnp.int32, (_C, _NP), 1)
    rcol = jax.lax.broadcasted_iota(jnp.int32, (_C, _RP), 1)

    def body(i, carry):
        s = s_ref[...]
        m = jnp.max(s, axis=1, keepdims=True)
        idx = jnp.min(jnp.where(s == m, col, _BIG_I), axis=1,
                      keepdims=True)
        ok = m > 0.0
        selmask = col == idx
        x1s = jnp.max(jnp.where(selmask, x1, -1e9), axis=1, keepdims=True)
        y1s = jnp.max(jnp.where(selmask, y1, -1e9), axis=1, keepdims=True)
        x2s = jnp.max(jnp.where(selmask, x2, -1e9), axis=1, keepdims=True)
        y2s = jnp.max(jnp.where(selmask, y2, -1e9), axis=1, keepdims=True)
        a_sel = jnp.maximum(x2s - x1s, 0.0) * jnp.maximum(y2s - y1s, 0.0)

        xx1 = jnp.maximum(x1s, x1)
        yy1 = jnp.maximum(y1s, y1)
        xx2 = jnp.minimum(x2s, x2)
        yy2 = jnp.minimum(y2s, y2)
        inter = jnp.maximum(xx2 - xx1, 0.0) * jnp.maximum(yy2 - yy1, 0.0)
        denom = a_sel + areas - inter + 1e-8
        sup = (inter > _IOU_THR * denom) | selmask
        s_ref[...] = jnp.where(sup & ok, -1.0, s)

        hit = rcol == i
        rec_s_ref[...] = jnp.where(hit, jnp.where(ok, m, -1.0), rec_s_ref[...])
        rec_i_ref[...] = jnp.where(hit, jnp.where(ok, idx, -1), rec_i_ref[...])
        return carry

    jax.lax.fori_loop(0, _MAX_DET, body, 0, unroll=False)

    out_s_ref[...] = jnp.full((1, _RP), -1.0, jnp.float32)
    out_l_ref[...] = jnp.full((1, _RP), -1, jnp.int32)
    out_b_ref[...] = jnp.full((4, _RP), -1.0, jnp.float32)

    fi = (jax.lax.broadcasted_iota(jnp.int32, (_C, _RP), 0) * _RP
          + jax.lax.broadcasted_iota(jnp.int32, (_C, _RP), 1))
    ocol = jax.lax.broadcasted_iota(jnp.int32, (1, _RP), 1)
    bcol = jax.lax.broadcasted_iota(jnp.int32, (1, _NP), 1)
    brow = jax.lax.broadcasted_iota(jnp.int32, (4, _NP), 1)

    def mbody(j, carry):
        rec = rec_s_ref[...]
        m2 = jnp.max(rec)
        fidx = jnp.min(jnp.where(rec == m2, fi, _BIG_I))
        kidx = jnp.max(jnp.where(fi == fidx, rec_i_ref[...], -2))
        label = fidx // _RP
        valid = kidx >= 0
        rec_s_ref[...] = jnp.where(fi == fidx, -2.0, rec)

        coords = jnp.max(jnp.where(brow == kidx, rows_ref[...], -1e9),
                         axis=1, keepdims=True)
        hit = ocol == j
        out_s_ref[...] = jnp.where(hit, jnp.where(valid, m2, -1.0),
                                   out_s_ref[...])
        out_l_ref[...] = jnp.where(hit, jnp.where(valid, label, -1),
                                   out_l_ref[...])
        out_b_ref[...] = jnp.where(hit, jnp.where(valid, coords, -1.0),
                                   out_b_ref[...])
        return carry

    jax.lax.fori_loop(0, _MAX_DET, mbody, 0, unroll=False)


@jax.jit
def kernel(boxes, classes):
    rows = jnp.zeros((4, _NP), jnp.float32).at[:, :_N].set(boxes.T)
    scores = jnp.full((_C, _NP), -1.0, jnp.float32).at[:, :_N].set(classes.T)

    out_s, out_l, out_b = pl.pallas_call(
        _nms_body,
        out_shape=[
            jax.ShapeDtypeStruct((1, _RP), jnp.float32),
            jax.ShapeDtypeStruct((1, _RP), jnp.int32),
            jax.ShapeDtypeStruct((4, _RP), jnp.float32),
        ],
        in_specs=[
            pl.BlockSpec(memory_space=pltpu.VMEM),
            pl.BlockSpec(memory_space=pltpu.VMEM),
        ],
        out_specs=[
            pl.BlockSpec(memory_space=pltpu.VMEM),
            pl.BlockSpec(memory_space=pltpu.VMEM),
            pl.BlockSpec(memory_space=pltpu.VMEM),
        ],
        scratch_shapes=[
            pltpu.VMEM((_C, _NP), jnp.float32),
            pltpu.VMEM((_C, _RP), jnp.float32),
            pltpu.VMEM((_C, _RP), jnp.int32),
        ],
    )(rows, scores)

    boxes_out = out_b.T[:_MAX_DET]
    scores_out = out_s[0, :_MAX_DET]
    labels_out = out_l[0, :_MAX_DET]
    return boxes_out, scores_out, labels_out


# speedup vs baseline: 7.1311x; 3.5060x over previous
import jax
import jax.numpy as jnp
from jax import lax
from jax.experimental import pallas as pl
from jax.experimental.pallas import tpu as pltpu
from jax.experimental.pallas import tpu_sc as plsc

_N = 20000
_C = 80
_MAX_DET = 300
_IOU_THR = 0.5
_SCORE_THR = 0.05
_NP = 20096
_RP = 304
_BUF = 512
_NB = _NP // 16
_KB = _RP // 16
_CB = _BUF // 16
_NW = 32
_BIG_I = 2**30


def _io16():
    return lax.broadcasted_iota(jnp.int32, (16,), 0)


def _vmaxsplat(v):
    return plsc.cummax(lax.rev(plsc.cummax(v), (0,)))


def _sc_nms(scores_hbm, rows_hbm, rec_s_hbm, rec_i_hbm,
            x1v, y1v, x2v, y2v, sv, csv, civ,
            cx1, cy1, cx2, cy2, car,
            kx1, ky1, kx2, ky2, kar, rsv, riv):
    wid = lax.axis_index("s") * 2 + lax.axis_index("c")
    io16 = _io16()
    fneg1 = jnp.full((16,), -1.0, jnp.float32)
    izero = jnp.zeros((16,), jnp.int32)

    pltpu.sync_copy(rows_hbm.at[0], x1v)
    pltpu.sync_copy(rows_hbm.at[1], y1v)
    pltpu.sync_copy(rows_hbm.at[2], x2v)
    pltpu.sync_copy(rows_hbm.at[3], y2v)

    def count3(t1, t2, t3):
        def body(i, carry):
            c1, c2, c3 = carry
            v = sv[pl.ds(i * 16, 16)]
            c1 = c1 + jnp.where(v > t1, 1, 0)
            c2 = c2 + jnp.where(v > t2, 1, 0)
            c3 = c3 + jnp.where(v > t3, 1, 0)
            return c1, c2, c3
        c1, c2, c3 = lax.fori_loop(0, _NB, body, (izero, izero, izero))
        return jnp.sum(c1), jnp.sum(c2), jnp.sum(c3)

    def choose_t(tcur):
        tlo0 = jnp.full((16,), _SCORE_THR, jnp.float32)

        def bcond(st):
            found, lo, hi, t, it = st
            return (found == 0) & (it < 24)

        def bbody(st):
            found, lo, hi, t, it = st
            span = hi - lo
            q1 = lo + span * 0.25
            q2 = lo + span * 0.5
            q3 = lo + span * 0.75
            collapsed = jnp.any(q1 <= lo) | jnp.any(q3 >= hi)
            f1, f2, f3 = count3(q1, q2, q3)
            le1 = f1 <= _BUF
            le2 = f2 <= _BUF
            le3 = f3 <= _BUF
            first_le_t = jnp.where(le1, q1, jnp.where(le2, q2, q3))
            f_first = jnp.where(le1, f1, jnp.where(le2, f2, f3))
            accept = le3 & (f_first >= 1)
            new_lo = jnp.where(~le3, q3,
                               jnp.where(~le2, q2, jnp.where(~le1, q1, lo)))
            new_hi = jnp.where(le3, first_le_t, hi)
            nfound = jnp.where(accept, 1, jnp.where(collapsed, 2, 0))
            nt = jnp.where(accept, first_le_t, new_lo)
            return nfound, new_lo, new_hi, nt, it + 1

        found, lo, hi, t, it = lax.while_loop(
            bcond, bbody, (jnp.int32(0), tlo0, tcur, tcur, jnp.int32(0)))
        return jnp.where(found == 1, t, lo)

    def process_class(c):
        pltpu.sync_copy(scores_hbm.at[c], sv)

        def init_body(i, carry):
            cnt, vmax = carry
            sl = pl.ds(i * 16, 16)
            v = sv[sl]
            m = v > _SCORE_THR
            v2 = jnp.where(m, v, -1.0)
            sv[sl] = v2
            return cnt + jnp.where(m, 1, 0), jnp.maximum(vmax, v2)
        cnt16, vmax16 = lax.fori_loop(0, _NB, init_body, (izero, fneg1))
        remaining0 = jnp.sum(cnt16)
        smax = _vmaxsplat(vmax16)

        for j in range(_KB):
            sl = pl.ds(j * 16, 16)
            kx1[sl] = jnp.full((16,), 1e30, jnp.float32)
            ky1[sl] = jnp.full((16,), 1e30, jnp.float32)
            kx2[sl] = jnp.full((16,), 1e30, jnp.float32)
            ky2[sl] = jnp.full((16,), 1e30, jnp.float32)
            kar[sl] = jnp.zeros((16,), jnp.float32)
            rsv[sl] = fneg1
            riv[sl] = jnp.full((16,), -1, jnp.int32)

        def refill_branch(st):
            kept, kp_prev, remaining, leftover, tcur, mv, pm0, pm1 = st
            t = lax.cond(
                leftover > 0,
                lambda _: tcur,
                lambda _: lax.cond(
                    remaining <= _BUF,
                    lambda __: jnp.full((16,), _SCORE_THR, jnp.float32),
                    lambda __: choose_t(tcur), 0),
                0)

            for j in range(_CB):
                csv[pl.ds(j * 16, 16)] = fneg1
                civ[pl.ds(j * 16, 16)] = izero

            def cbody(i, carry):
                stored, matches = carry
                sl = pl.ds(i * 16, 16)
                v = sv[sl]
                m = v > t
                mi = jnp.where(m, 1, 0)
                dest = stored + plsc.cumsum(mi) - 1
                okm = m & (dest < _BUF)
                destc = jnp.minimum(jnp.maximum(dest, 0), _BUF - 1)
                plsc.store_scatter(csv, [destc], v, mask=okm)
                plsc.store_scatter(civ, [destc], i * 16 + io16, mask=okm)
                sv[sl] = jnp.where(okm, -1.0, v)
                return (stored + jnp.sum(jnp.where(okm, 1, 0)),
                        matches + jnp.sum(mi))
            stored, matches = lax.fori_loop(0, _NB, cbody,
                                            (jnp.int32(0), jnp.int32(0)))

            def gbody(i, carry):
                pm0, pm1 = carry
                sl = pl.ds(i * 16, 16)
                idx = civ[sl]
                a = plsc.load_gather(x1v, [idx])
                b = plsc.load_gather(y1v, [idx])
                d = plsc.load_gather(x2v, [idx])
                e = plsc.load_gather(y2v, [idx])
                cx1[sl] = a
                cy1[sl] = b
                cx2[sl] = d
                cy2[sl] = e
                car[sl] = jnp.maximum(d - a, 0.0) * jnp.maximum(e - b, 0.0)
                vm = _vmaxsplat(csv[sl])
                pm0 = jnp.where(io16 == i, vm, pm0)
                pm1 = jnp.where(io16 == (i - 16), vm, pm1)
                return pm0, pm1
            pm0, pm1 = lax.fori_loop(0, _CB, gbody, (fneg1, fneg1))

            bail = stored == 0
            remaining2 = jnp.where(bail, 0, remaining - stored)
            leftover2 = jnp.where(bail, 0, matches - stored)
            mv2 = _vmaxsplat(jnp.maximum(pm0, pm1))
            return (kept, kept, remaining2, leftover2, t, mv2, pm0, pm1)

        def pick_branch(st):
            kept, kp_prev, remaining, leftover, tcur, mv, pm0, pm1 = st
            ffs0 = jnp.max(plsc.all_reduce_ffs(pm0 >= mv))
            ffs1 = jnp.max(plsc.all_reduce_ffs(pm1 >= mv))
            j = jnp.where(ffs0 < 16, ffs0, ffs1 + 16)
            sl = pl.ds(j * 16, 16)
            l = jnp.max(plsc.all_reduce_ffs(csv[sl] >= mv))
            pos = j * 16 + l
            lane_eq = io16 == l

            def ext(ref):
                return _vmaxsplat(jnp.where(lane_eq, ref[sl], -3e38))
            bx1 = ext(cx1)
            by1 = ext(cy1)
            bx2 = ext(cx2)
            by2 = ext(cy2)
            ba = ext(car)
            bidx = jnp.max(jnp.where(lane_eq, civ[sl], -1))

            nkv = (kp_prev + 15) // 16

            def kbody(k, acc):
                slk = pl.ds(k * 16, 16)
                xx1 = jnp.maximum(kx1[slk], bx1)
                yy1 = jnp.maximum(ky1[slk], by1)
                xx2 = jnp.minimum(kx2[slk], bx2)
                yy2 = jnp.minimum(ky2[slk], by2)
                inter = (jnp.maximum(xx2 - xx1, 0.0)
                         * jnp.maximum(yy2 - yy1, 0.0))
                return acc | (inter > _IOU_THR * (kar[slk] + ba - inter + 1e-8))
            accv = lax.fori_loop(0, nkv, kbody, jnp.zeros((16,), jnp.bool_))
            sup = jnp.any(accv)

            posv = jnp.full((16,), pos, jnp.int32)
            m0 = io16 == 0

            def sup_fn(args):
                pm0, pm1, kept = args
                plsc.store_scatter(csv, [posv], fneg1, mask=m0)
                nv = _vmaxsplat(csv[sl])
                pm0 = jnp.where(io16 == j, nv, pm0)
                pm1 = jnp.where(io16 == (j - 16), nv, pm1)
                return pm0, pm1, kept

            def keep_fn(args):
                pm0, pm1, kept = args
                kiv = jnp.full((16,), kept, jnp.int32)
                plsc.store_scatter(kx1, [kiv], bx1, mask=m0)
                plsc.store_scatter(ky1, [kiv], by1, mask=m0)
                plsc.store_scatter(kx2, [kiv], bx2, mask=m0)
                plsc.store_scatter(ky2, [kiv], by2, mask=m0)
                plsc.store_scatter(kar, [kiv], ba, mask=m0)
                plsc.store_scatter(rsv, [kiv], mv, mask=m0)
                plsc.store_scatter(riv, [kiv],
                                   jnp.full((16,), bidx, jnp.int32), mask=m0)

                def sbody(jj, carry):
                    pm0, pm1 = carry
                    slj = pl.ds(jj * 16, 16)
                    cs = csv[slj]
                    xx1 = jnp.maximum(cx1[slj], bx1)
                    yy1 = jnp.maximum(cy1[slj], by1)
                    xx2 = jnp.minimum(cx2[slj], bx2)
                    yy2 = jnp.minimum(cy2[slj], by2)
                    inter = (jnp.maximum(xx2 - xx1, 0.0)
                             * jnp.maximum(yy2 - yy1, 0.0))
                    bad = ((inter > _IOU_THR * (car[slj] + ba - inter + 1e-8))
                           | ((jj * 16 + io16) == pos))
                    ncs = jnp.where(bad, -1.0, cs)
                    csv[slj] = ncs
                    nv = _vmaxsplat(ncs)
                    pm0 = jnp.where(io16 == jj, nv, pm0)
                    pm1 = jnp.where(io16 == (jj - 16), nv, pm1)
                    return pm0, pm1
                pm0, pm1 = lax.fori_loop(0, _CB, sbody, (pm0, pm1))
                return pm0, pm1, kept + 1

            pm0, pm1, kept = lax.cond(sup, sup_fn, keep_fn, (pm0, pm1, kept))
            mv2 = _vmaxsplat(jnp.maximum(pm0, pm1))
            return (kept, kp_prev, remaining, leftover, tcur, mv2, pm0, pm1)

        def wcond(st):
            kept, kp_prev, remaining, leftover, tcur, mv, pm0, pm1 = st
            return (kept < _MAX_DET) & (jnp.any(mv > 0.0) | (remaining > 0))

        def wbody(st):
            need_refill = ~jnp.any(st[5] > 0.0)
            return lax.cond(need_refill, refill_branch, pick_branch, st)

        st0 = (jnp.int32(0), jnp.int32(0), remaining0, jnp.int32(0),
               smax, jnp.zeros((16,), jnp.float32), fneg1, fneg1)
        lax.while_loop(wcond, wbody, st0)

        pltpu.sync_copy(rsv, rec_s_hbm.at[c])
        pltpu.sync_copy(riv, rec_i_hbm.at[c])

    for k in range(3):
        c = wid + _NW * k

        @pl.when(c < _C)
        def _():
            process_class(c)


def _merge_body(rows_ref, rec_s_ref, rec_i_ref,
                out_s_ref, out_l_ref, out_b_ref, ms_ref):
    ms_ref[...] = rec_s_ref[...]
    out_s_ref[...] = jnp.full((1, _RP), -1.0, jnp.float32)
    out_l_ref[...] = jnp.full((1, _RP), -1, jnp.int32)
    out_b_ref[...] = jnp.full((4, _RP), -1.0, jnp.float32)

    fi = (jax.lax.broadcasted_iota(jnp.int32, (_C, _RP), 0) * _RP
          + jax.lax.broadcasted_iota(jnp.int32, (_C, _RP), 1))
    ocol = jax.lax.broadcasted_iota(jnp.int32, (1, _RP), 1)
    brow = jax.lax.broadcasted_iota(jnp.int32, (4, _NP), 1)

    def mbody(j, carry):
        rec = ms_ref[...]
        m2 = jnp.max(rec)
        fidx = jnp.min(jnp.where(rec == m2, fi, _BIG_I))
        kidx = jnp.max(jnp.where(fi == fidx, rec_i_ref[...], -2))
        label = fidx // _RP
        valid = kidx >= 0
        ms_ref[...] = jnp.where(fi == fidx, -2.0, rec)

        coords = jnp.max(jnp.where(brow == kidx, rows_ref[...], -1e9),
                         axis=1, keepdims=True)
        hit = ocol == j
        out_s_ref[...] = jnp.where(hit, jnp.where(valid, m2, -1.0),
                                   out_s_ref[...])
        out_l_ref[...] = jnp.where(hit, jnp.where(valid, label, -1),
                                   out_l_ref[...])
        out_b_ref[...] = jnp.where(hit, jnp.where(valid, coords, -1.0),
                                   out_b_ref[...])
        return carry

    jax.lax.fori_loop(0, _MAX_DET, mbody, 0, unroll=False)


@jax.jit
def kernel(boxes, classes):
    rows = jnp.zeros((4, _NP), jnp.float32).at[:, :_N].set(boxes.T)
    scores_t = jnp.full((_C, _NP), -1.0, jnp.float32).at[:, :_N].set(
        classes.T)

    mesh = plsc.VectorSubcoreMesh(core_axis_name="c", subcore_axis_name="s")
    rec_s, rec_i = pl.kernel(
        _sc_nms,
        out_type=[
            jax.ShapeDtypeStruct((_C, _RP), jnp.float32),
            jax.ShapeDtypeStruct((_C, _RP), jnp.int32),
        ],
        mesh=mesh,
        compiler_params=pltpu.CompilerParams(needs_layout_passes=False),
        scratch_types=[
            pltpu.VMEM((_NP,), jnp.float32),
            pltpu.VMEM((_NP,), jnp.float32),
            pltpu.VMEM((_NP,), jnp.float32),
            pltpu.VMEM((_NP,), jnp.float32),
            pltpu.VMEM((_NP,), jnp.float32),
            pltpu.VMEM((_BUF,), jnp.float32),
            pltpu.VMEM((_BUF,), jnp.int32),
            pltpu.VMEM((_BUF,), jnp.float32),
            pltpu.VMEM((_BUF,), jnp.float32),
            pltpu.VMEM((_BUF,), jnp.float32),
            pltpu.VMEM((_BUF,), jnp.float32),
            pltpu.VMEM((_BUF,), jnp.float32),
            pltpu.VMEM((_RP,), jnp.float32),
            pltpu.VMEM((_RP,), jnp.float32),
            pltpu.VMEM((_RP,), jnp.float32),
            pltpu.VMEM((_RP,), jnp.float32),
            pltpu.VMEM((_RP,), jnp.float32),
            pltpu.VMEM((_RP,), jnp.float32),
            pltpu.VMEM((_RP,), jnp.int32),
        ],
    )(scores_t, rows)

    out_s, out_l, out_b = pl.pallas_call(
        _merge_body,
        out_shape=[
            jax.ShapeDtypeStruct((1, _RP), jnp.float32),
            jax.ShapeDtypeStruct((1, _RP), jnp.int32),
            jax.ShapeDtypeStruct((4, _RP), jnp.float32),
        ],
        in_specs=[
            pl.BlockSpec(memory_space=pltpu.VMEM),
            pl.BlockSpec(memory_space=pltpu.VMEM),
            pl.BlockSpec(memory_space=pltpu.VMEM),
        ],
        out_specs=[
            pl.BlockSpec(memory_space=pltpu.VMEM),
            pl.BlockSpec(memory_space=pltpu.VMEM),
            pl.BlockSpec(memory_space=pltpu.VMEM),
        ],
        scratch_shapes=[
            pltpu.VMEM((_C, _RP), jnp.float32),
        ],
    )(rows, rec_s, rec_i)

    boxes_out = out_b.T[:_MAX_DET]
    scores_out = out_s[0, :_MAX_DET]
    labels_out = out_l[0, :_MAX_DET]
    return boxes_out, scores_out, labels_out


# speedup vs baseline: 7.8633x; 1.1027x over previous
import jax
import jax.numpy as jnp
from jax import lax
from jax.experimental import pallas as pl
from jax.experimental.pallas import tpu as pltpu
from jax.experimental.pallas import tpu_sc as plsc

_N = 20000
_C = 80
_MAX_DET = 300
_IOU_THR = 0.5
_SCORE_THR = 0.05
_NP = 20096
_RP = 304
_BUF = 512
_NB = _NP // 16
_KB = _RP // 16
_CB = _BUF // 16
_NW = 32
_BIG_I = 2**30


def _io16():
    return lax.broadcasted_iota(jnp.int32, (16,), 0)


def _vmaxsplat(v):
    return plsc.cummax(lax.rev(plsc.cummax(v), (0,)))


def _sc_nms(scores_hbm, rows_hbm, rec_s_hbm, rec_i_hbm,
            x1v, y1v, x2v, y2v, sv, csv, civ,
            cx1, cy1, cx2, cy2, car,
            kx1, ky1, kx2, ky2, kar, rsv, riv):
    wid = lax.axis_index("s") * 2 + lax.axis_index("c")
    io16 = _io16()
    fneg1 = jnp.full((16,), -1.0, jnp.float32)
    izero = jnp.zeros((16,), jnp.int32)

    pltpu.sync_copy(rows_hbm.at[0], x1v)
    pltpu.sync_copy(rows_hbm.at[1], y1v)
    pltpu.sync_copy(rows_hbm.at[2], x2v)
    pltpu.sync_copy(rows_hbm.at[3], y2v)

    def count3(t1, t2, t3):
        def body(i, carry):
            c1, c2, c3 = carry
            v = sv[pl.ds(i * 16, 16)]
            c1 = c1 + jnp.where(v > t1, 1, 0)
            c2 = c2 + jnp.where(v > t2, 1, 0)
            c3 = c3 + jnp.where(v > t3, 1, 0)
            return c1, c2, c3
        c1, c2, c3 = lax.fori_loop(0, _NB, body, (izero, izero, izero))
        return jnp.sum(c1), jnp.sum(c2), jnp.sum(c3)

    def choose_t(tcur):
        tlo0 = jnp.full((16,), _SCORE_THR, jnp.float32)

        def bcond(st):
            found, lo, hi, t, it = st
            return (found == 0) & (it < 24)

        def bbody(st):
            found, lo, hi, t, it = st
            span = hi - lo
            q1 = lo + span * 0.25
            q2 = lo + span * 0.5
            q3 = lo + span * 0.75
            collapsed = jnp.any(q1 <= lo) | jnp.any(q3 >= hi)
            f1, f2, f3 = count3(q1, q2, q3)
            le1 = f1 <= _BUF
            le2 = f2 <= _BUF
            le3 = f3 <= _BUF
            first_le_t = jnp.where(le1, q1, jnp.where(le2, q2, q3))
            f_first = jnp.where(le1, f1, jnp.where(le2, f2, f3))
            accept = le3 & (f_first >= 1)
            new_lo = jnp.where(~le3, q3,
                               jnp.where(~le2, q2, jnp.where(~le1, q1, lo)))
            new_hi = jnp.where(le3, first_le_t, hi)
            nfound = jnp.where(accept, 1, jnp.where(collapsed, 2, 0))
            nt = jnp.where(accept, first_le_t, new_lo)
            return nfound, new_lo, new_hi, nt, it + 1

        found, lo, hi, t, it = lax.while_loop(
            bcond, bbody, (jnp.int32(0), tlo0, tcur, tcur, jnp.int32(0)))
        return jnp.where(found == 1, t, lo)

    def process_class(c):
        pltpu.sync_copy(scores_hbm.at[c], sv)

        def init_body(i, carry):
            cnt, vmax = carry
            sl = pl.ds(i * 16, 16)
            v = sv[sl]
            m = v > _SCORE_THR
            v2 = jnp.where(m, v, -1.0)
            sv[sl] = v2
            return cnt + jnp.where(m, 1, 0), jnp.maximum(vmax, v2)
        cnt16, vmax16 = lax.fori_loop(0, _NB, init_body, (izero, fneg1))
        remaining0 = jnp.sum(cnt16)
        smax = _vmaxsplat(vmax16)

        for j in range(_KB):
            sl = pl.ds(j * 16, 16)
            kx1[sl] = jnp.full((16,), 1e30, jnp.float32)
            ky1[sl] = jnp.full((16,), 1e30, jnp.float32)
            kx2[sl] = jnp.full((16,), 1e30, jnp.float32)
            ky2[sl] = jnp.full((16,), 1e30, jnp.float32)
            kar[sl] = jnp.zeros((16,), jnp.float32)
            rsv[sl] = fneg1
            riv[sl] = jnp.full((16,), -1, jnp.int32)

        def fold_acc(j, cs, acc, accj):
            m = cs > acc
            return jnp.where(m, cs, acc), jnp.where(m, j, accj)

        def refill_branch(st):
            kept, kp_prev, remaining, leftover, tcur, acc, accj = st
            t = lax.cond(
                leftover > 0,
                lambda _: tcur,
                lambda _: lax.cond(
                    remaining <= _BUF,
                    lambda __: jnp.full((16,), _SCORE_THR, jnp.float32),
                    lambda __: choose_t(tcur), 0),
                0)

            for j in range(_CB):
                csv[pl.ds(j * 16, 16)] = fneg1
                civ[pl.ds(j * 16, 16)] = izero

            def cbody(i, carry):
                stored, matches = carry
                sl = pl.ds(i * 16, 16)
                v = sv[sl]
                m = v > t
                mi = jnp.where(m, 1, 0)
                dest = stored + plsc.cumsum(mi) - 1
                okm = m & (dest < _BUF)
                destc = jnp.minimum(jnp.maximum(dest, 0), _BUF - 1)
                plsc.store_scatter(csv, [destc], v, mask=okm)
                plsc.store_scatter(civ, [destc], i * 16 + io16, mask=okm)
                sv[sl] = jnp.where(okm, -1.0, v)
                return (stored + jnp.sum(jnp.where(okm, 1, 0)),
                        matches + jnp.sum(mi))
            stored, matches = lax.fori_loop(0, _NB, cbody,
                                            (jnp.int32(0), jnp.int32(0)))

            def gbody(i, carry):
                acc, accj = carry
                sl = pl.ds(i * 16, 16)
                idx = civ[sl]
                a = plsc.load_gather(x1v, [idx])
                b = plsc.load_gather(y1v, [idx])
                d = plsc.load_gather(x2v, [idx])
                e = plsc.load_gather(y2v, [idx])
                cx1[sl] = a
                cy1[sl] = b
                cx2[sl] = d
                cy2[sl] = e
                car[sl] = jnp.maximum(d - a, 0.0) * jnp.maximum(e - b, 0.0)
                return fold_acc(i, csv[sl], acc, accj)
            acc, accj = lax.fori_loop(0, _CB, gbody, (fneg1, izero))

            bail = stored == 0
            remaining2 = jnp.where(bail, 0, remaining - stored)
            leftover2 = jnp.where(bail, 0, matches - stored)
            return (kept, kept, remaining2, leftover2, t, acc, accj)

        def pick_branch(st):
            kept, kp_prev, remaining, leftover, tcur, acc, accj = st
            mvs = jnp.max(acc)
            pos = jnp.min(jnp.where(acc >= mvs,
                                    accj * 16 + io16, _BIG_I))
            posv = jnp.full((16,), pos, jnp.int32)
            mv = jnp.full((16,), mvs, jnp.float32)
            bx1 = plsc.load_gather(cx1, [posv])
            by1 = plsc.load_gather(cy1, [posv])
            bx2 = plsc.load_gather(cx2, [posv])
            by2 = plsc.load_gather(cy2, [posv])
            ba = plsc.load_gather(car, [posv])
            bidxv = plsc.load_gather(civ, [posv])

            nkv = (kp_prev + 15) // 16

            def kbody(k, sacc):
                slk = pl.ds(k * 16, 16)
                xx1 = jnp.maximum(kx1[slk], bx1)
                yy1 = jnp.maximum(ky1[slk], by1)
                xx2 = jnp.minimum(kx2[slk], bx2)
                yy2 = jnp.minimum(ky2[slk], by2)
                inter = (jnp.maximum(xx2 - xx1, 0.0)
                         * jnp.maximum(yy2 - yy1, 0.0))
                return sacc | (inter > _IOU_THR * (kar[slk] + ba - inter
                                                   + 1e-8))
            accv = lax.fori_loop(0, nkv, kbody, jnp.zeros((16,), jnp.bool_))
            sup = jnp.max(jnp.where(accv, 1, 0)) > 0

            m0 = io16 == 0

            def sup_fn(args):
                acc, accj, kept = args
                plsc.store_scatter(csv, [posv], fneg1, mask=m0)

                def rbody(jj, carry):
                    acc, accj = carry
                    return fold_acc(jj, csv[pl.ds(jj * 16, 16)], acc, accj)
                acc, accj = lax.fori_loop(0, _CB, rbody, (fneg1, izero))
                return acc, accj, kept

            def keep_fn(args):
                acc, accj, kept = args
                kiv = jnp.full((16,), kept, jnp.int32)
                plsc.store_scatter(kx1, [kiv], bx1, mask=m0)
                plsc.store_scatter(ky1, [kiv], by1, mask=m0)
                plsc.store_scatter(kx2, [kiv], bx2, mask=m0)
                plsc.store_scatter(ky2, [kiv], by2, mask=m0)
                plsc.store_scatter(kar, [kiv], ba, mask=m0)
                plsc.store_scatter(rsv, [kiv], mv, mask=m0)
                plsc.store_scatter(riv, [kiv], bidxv, mask=m0)
                plsc.store_scatter(csv, [posv], fneg1, mask=m0)

                def sbody(jj, carry):
                    acc, accj = carry
                    slj = pl.ds(jj * 16, 16)
                    cs = csv[slj]
                    xx1 = jnp.maximum(cx1[slj], bx1)
                    yy1 = jnp.maximum(cy1[slj], by1)
                    xx2 = jnp.minimum(cx2[slj], bx2)
                    yy2 = jnp.minimum(cy2[slj], by2)
                    inter = (jnp.maximum(xx2 - xx1, 0.0)
                             * jnp.maximum(yy2 - yy1, 0.0))
                    bad = inter > _IOU_THR * (car[slj] + ba - inter + 1e-8)
                    ncs = jnp.where(bad, -1.0, cs)
                    csv[slj] = ncs
                    return fold_acc(jj, ncs, acc, accj)
                acc, accj = lax.fori_loop(0, _CB, sbody, (fneg1, izero))
                return acc, accj, kept + 1

            acc, accj, kept = lax.cond(sup, sup_fn, keep_fn,
                                       (acc, accj, kept))
            return (kept, kp_prev, remaining, leftover, tcur, acc, accj)

        def wcond(st):
            kept, kp_prev, remaining, leftover, tcur, acc, accj = st
            return (kept < _MAX_DET) & ((jnp.max(acc) > 0.0)
                                        | (remaining > 0))

        def wbody(st):
            need_refill = jnp.max(st[5]) <= 0.0
            return lax.cond(need_refill, refill_branch, pick_branch, st)

        st0 = (jnp.int32(0), jnp.int32(0), remaining0, jnp.int32(0),
               smax, fneg1, izero)
        lax.while_loop(wcond, wbody, st0)

        pltpu.sync_copy(rsv, rec_s_hbm.at[c])
        pltpu.sync_copy(riv, rec_i_hbm.at[c])

    for k in range(3):
        c = wid + _NW * k

        @pl.when(c < _C)
        def _():
            process_class(c)


def _merge_body(rows_ref, rec_s_ref, rec_i_ref,
                out_s_ref, out_l_ref, out_b_ref, ms_ref):
    ms_ref[...] = rec_s_ref[...]
    out_s_ref[...] = jnp.full((1, _RP), -1.0, jnp.float32)
    out_l_ref[...] = jnp.full((1, _RP), -1, jnp.int32)
    out_b_ref[...] = jnp.full((4, _RP), -1.0, jnp.float32)

    fi = (jax.lax.broadcasted_iota(jnp.int32, (_C, _RP), 0) * _RP
          + jax.lax.broadcasted_iota(jnp.int32, (_C, _RP), 1))
    ocol = jax.lax.broadcasted_iota(jnp.int32, (1, _RP), 1)
    brow = jax.lax.broadcasted_iota(jnp.int32, (4, _NP), 1)

    def mbody(j, carry):
        rec = ms_ref[...]
        m2 = jnp.max(rec)
        fidx = jnp.min(jnp.where(rec == m2, fi, _BIG_I))
        kidx = jnp.max(jnp.where(fi == fidx, rec_i_ref[...], -2))
        label = fidx // _RP
        valid = kidx >= 0
        ms_ref[...] = jnp.where(fi == fidx, -2.0, rec)

        coords = jnp.max(jnp.where(brow == kidx, rows_ref[...], -1e9),
                         axis=1, keepdims=True)
        hit = ocol == j
        out_s_ref[...] = jnp.where(hit, jnp.where(valid, m2, -1.0),
                                   out_s_ref[...])
        out_l_ref[...] = jnp.where(hit, jnp.where(valid, label, -1),
                                   out_l_ref[...])
        out_b_ref[...] = jnp.where(hit, jnp.where(valid, coords, -1.0),
                                   out_b_ref[...])
        return carry

    jax.lax.fori_loop(0, _MAX_DET, mbody, 0, unroll=False)


@jax.jit
def kernel(boxes, classes):
    rows = jnp.zeros((4, _NP), jnp.float32).at[:, :_N].set(boxes.T)
    scores_t = jnp.full((_C, _NP), -1.0, jnp.float32).at[:, :_N].set(
        classes.T)

    mesh = plsc.VectorSubcoreMesh(core_axis_name="c", subcore_axis_name="s")
    rec_s, rec_i = pl.kernel(
        _sc_nms,
        out_type=[
            jax.ShapeDtypeStruct((_C, _RP), jnp.float32),
            jax.ShapeDtypeStruct((_C, _RP), jnp.int32),
        ],
        mesh=mesh,
        compiler_params=pltpu.CompilerParams(needs_layout_passes=False),
        scratch_types=[
            pltpu.VMEM((_NP,), jnp.float32),
            pltpu.VMEM((_NP,), jnp.float32),
            pltpu.VMEM((_NP,), jnp.float32),
            pltpu.VMEM((_NP,), jnp.float32),
            pltpu.VMEM((_NP,), jnp.float32),
            pltpu.VMEM((_BUF,), jnp.float32),
            pltpu.VMEM((_BUF,), jnp.int32),
            pltpu.VMEM((_BUF,), jnp.float32),
            pltpu.VMEM((_BUF,), jnp.float32),
            pltpu.VMEM((_BUF,), jnp.float32),
            pltpu.VMEM((_BUF,), jnp.float32),
            pltpu.VMEM((_BUF,), jnp.float32),
            pltpu.VMEM((_RP,), jnp.float32),
            pltpu.VMEM((_RP,), jnp.float32),
            pltpu.VMEM((_RP,), jnp.float32),
            pltpu.VMEM((_RP,), jnp.float32),
            pltpu.VMEM((_RP,), jnp.float32),
            pltpu.VMEM((_RP,), jnp.float32),
            pltpu.VMEM((_RP,), jnp.int32),
        ],
    )(scores_t, rows)

    out_s, out_l, out_b = pl.pallas_call(
        _merge_body,
        out_shape=[
            jax.ShapeDtypeStruct((1, _RP), jnp.float32),
            jax.ShapeDtypeStruct((1, _RP), jnp.int32),
            jax.ShapeDtypeStruct((4, _RP), jnp.float32),
        ],
        in_specs=[
            pl.BlockSpec(memory_space=pltpu.VMEM),
            pl.BlockSpec(memory_space=pltpu.VMEM),
            pl.BlockSpec(memory_space=pltpu.VMEM),
        ],
        out_specs=[
            pl.BlockSpec(memory_space=pltpu.VMEM),
            pl.BlockSpec(memory_space=pltpu.VMEM),
            pl.BlockSpec(memory_space=pltpu.VMEM),
        ],
        scratch_shapes=[
            pltpu.VMEM((_C, _RP), jnp.float32),
        ],
    )(rows, rec_s, rec_i)

    boxes_out = out_b.T[:_MAX_DET]
    scores_out = out_s[0, :_MAX_DET]
    labels_out = out_l[0, :_MAX_DET]
    return boxes_out, scores_out, labels_out


# speedup vs baseline: 9.6674x; 1.2294x over previous
import jax
import jax.numpy as jnp
from jax import lax
from jax.experimental import pallas as pl
from jax.experimental.pallas import tpu as pltpu
from jax.experimental.pallas import tpu_sc as plsc

_N = 20000
_C = 80
_MAX_DET = 300
_IOU_THR = 0.5
_SCORE_THR = 0.05
_NP = 20096
_RP = 304
_BUF = 512
_NB = _NP // 16
_KB = _RP // 16
_CB = _BUF // 16
_NW = 32
_BIG_I = 2**30


def _io16():
    return lax.broadcasted_iota(jnp.int32, (16,), 0)


def _vmaxsplat(v):
    return plsc.cummax(lax.rev(plsc.cummax(v), (0,)))


def _sc_nms(scores_hbm, rows_hbm, rec_s_hbm, rec_i_hbm,
            x1v, y1v, x2v, y2v, sv, csv, civ,
            cx1, cy1, cx2, cy2, car, chc,
            kx1, ky1, kx2, ky2, kar, rsv, riv):
    wid = lax.axis_index("s") * 2 + lax.axis_index("c")
    io16 = _io16()
    fneg1 = jnp.full((16,), -1.0, jnp.float32)
    izero = jnp.zeros((16,), jnp.int32)

    pltpu.sync_copy(rows_hbm.at[0], x1v)
    pltpu.sync_copy(rows_hbm.at[1], y1v)
    pltpu.sync_copy(rows_hbm.at[2], x2v)
    pltpu.sync_copy(rows_hbm.at[3], y2v)

    def count3(t1, t2, t3):
        def body(i, carry):
            c1, c2, c3 = carry
            v = sv[pl.ds(i * 16, 16)]
            c1 = c1 + jnp.where(v > t1, 1, 0)
            c2 = c2 + jnp.where(v > t2, 1, 0)
            c3 = c3 + jnp.where(v > t3, 1, 0)
            return c1, c2, c3
        c1, c2, c3 = lax.fori_loop(0, _NB, body, (izero, izero, izero))
        return jnp.sum(c1), jnp.sum(c2), jnp.sum(c3)

    def choose_t(tcur):
        tlo0 = jnp.full((16,), _SCORE_THR, jnp.float32)

        def bcond(st):
            found, lo, hi, t, it = st
            return (found == 0) & (it < 24)

        def bbody(st):
            found, lo, hi, t, it = st
            span = hi - lo
            q1 = lo + span * 0.25
            q2 = lo + span * 0.5
            q3 = lo + span * 0.75
            collapsed = jnp.any(q1 <= lo) | jnp.any(q3 >= hi)
            f1, f2, f3 = count3(q1, q2, q3)
            le1 = f1 <= _BUF
            le2 = f2 <= _BUF
            le3 = f3 <= _BUF
            first_le_t = jnp.where(le1, q1, jnp.where(le2, q2, q3))
            f_first = jnp.where(le1, f1, jnp.where(le2, f2, f3))
            accept = le3 & (f_first >= 1)
            new_lo = jnp.where(~le3, q3,
                               jnp.where(~le2, q2, jnp.where(~le1, q1, lo)))
            new_hi = jnp.where(le3, first_le_t, hi)
            nfound = jnp.where(accept, 1, jnp.where(collapsed, 2, 0))
            nt = jnp.where(accept, first_le_t, new_lo)
            return nfound, new_lo, new_hi, nt, it + 1

        found, lo, hi, t, it = lax.while_loop(
            bcond, bbody, (jnp.int32(0), tlo0, tcur, tcur, jnp.int32(0)))
        return jnp.where(found == 1, t, lo)

    def process_class(c):
        pltpu.sync_copy(scores_hbm.at[c], sv)

        def init_body(i, carry):
            cnt, vmax = carry
            sl = pl.ds(i * 16, 16)
            v = sv[sl]
            m = v > _SCORE_THR
            v2 = jnp.where(m, v, -1.0)
            sv[sl] = v2
            return cnt + jnp.where(m, 1, 0), jnp.maximum(vmax, v2)
        cnt16, vmax16 = lax.fori_loop(0, _NB, init_body, (izero, fneg1))
        remaining0 = jnp.sum(cnt16)
        smax = _vmaxsplat(vmax16)

        for j in range(_KB):
            sl = pl.ds(j * 16, 16)
            kx1[sl] = jnp.full((16,), 1e30, jnp.float32)
            ky1[sl] = jnp.full((16,), 1e30, jnp.float32)
            kx2[sl] = jnp.full((16,), 1e30, jnp.float32)
            ky2[sl] = jnp.full((16,), 1e30, jnp.float32)
            kar[sl] = jnp.zeros((16,), jnp.float32)
            rsv[sl] = fneg1
            riv[sl] = jnp.full((16,), -1, jnp.int32)

        def fold_acc(j, cs, acc, accj):
            m = cs > acc
            return jnp.where(m, cs, acc), jnp.where(m, j, accj)

        def refill_branch(st):
            kept, kp_prev, remaining, leftover, tcur, acc, accj, nb = st
            t = lax.cond(
                leftover > 0,
                lambda _: tcur,
                lambda _: lax.cond(
                    remaining <= _BUF,
                    lambda __: jnp.full((16,), _SCORE_THR, jnp.float32),
                    lambda __: choose_t(tcur), 0),
                0)

            for j in range(_CB):
                csv[pl.ds(j * 16, 16)] = fneg1
                civ[pl.ds(j * 16, 16)] = izero

            def cbody(i, carry):
                stored, matches = carry
                sl = pl.ds(i * 16, 16)
                v = sv[sl]
                m = v > t
                mi = jnp.where(m, 1, 0)
                dest = stored + plsc.cumsum(mi) - 1
                okm = m & (dest < _BUF)
                destc = jnp.minimum(jnp.maximum(dest, 0), _BUF - 1)
                plsc.store_scatter(csv, [destc], v, mask=okm)
                plsc.store_scatter(civ, [destc], i * 16 + io16, mask=okm)
                sv[sl] = jnp.where(okm, -1.0, v)
                return (stored + jnp.sum(jnp.where(okm, 1, 0)),
                        matches + jnp.sum(mi))
            stored, matches = lax.fori_loop(0, _NB, cbody,
                                            (jnp.int32(0), jnp.int32(0)))

            nb2 = (stored + 15) // 16

            def gbody(i, carry):
                acc, accj = carry
                sl = pl.ds(i * 16, 16)
                idx = civ[sl]
                a = plsc.load_gather(x1v, [idx])
                b = plsc.load_gather(y1v, [idx])
                d = plsc.load_gather(x2v, [idx])
                e = plsc.load_gather(y2v, [idx])
                cx1[sl] = a
                cy1[sl] = b
                cx2[sl] = d
                cy2[sl] = e
                ar = jnp.maximum(d - a, 0.0) * jnp.maximum(e - b, 0.0)
                car[sl] = ar
                chc[sl] = _IOU_THR * (ar + 1e-8)
                return fold_acc(i, csv[sl], acc, accj)
            acc, accj = lax.fori_loop(0, nb2, gbody, (fneg1, izero))

            bail = stored == 0
            remaining2 = jnp.where(bail, 0, remaining - stored)
            leftover2 = jnp.where(bail, 0, matches - stored)
            return (kept, kept, remaining2, leftover2, t, acc, accj, nb2)

        def pick_branch(st):
            kept, kp_prev, remaining, leftover, tcur, acc, accj, nb = st
            mvs = jnp.max(acc)
            pos = jnp.min(jnp.where(acc >= mvs,
                                    accj * 16 + io16, _BIG_I))
            posv = jnp.full((16,), pos, jnp.int32)
            mv = jnp.full((16,), mvs, jnp.float32)
            bx1 = plsc.load_gather(cx1, [posv])
            by1 = plsc.load_gather(cy1, [posv])
            bx2 = plsc.load_gather(cx2, [posv])
            by2 = plsc.load_gather(cy2, [posv])
            bhc = plsc.load_gather(chc, [posv])
            bah = _IOU_THR * plsc.load_gather(car, [posv])
            bidxv = plsc.load_gather(civ, [posv])

            nkv = (kp_prev + 15) // 16

            def kbody(k, sacc):
                slk = pl.ds(k * 16, 16)
                xx1 = jnp.maximum(kx1[slk], bx1)
                yy1 = jnp.maximum(ky1[slk], by1)
                xx2 = jnp.minimum(kx2[slk], bx2)
                yy2 = jnp.minimum(ky2[slk], by2)
                inter = (jnp.maximum(xx2 - xx1, 0.0)
                         * jnp.maximum(yy2 - yy1, 0.0))
                return sacc | (inter * (1.0 + _IOU_THR) > kar[slk] + bah)
            accv = lax.fori_loop(0, nkv, kbody, jnp.zeros((16,), jnp.bool_))
            sup = jnp.max(jnp.where(accv, 1, 0)) > 0

            m0 = io16 == 0
            ineg1 = jnp.full((16,), -1, jnp.int32)

            def sup_fn(args):
                acc, accj, kept, nb = args
                plsc.store_scatter(csv, [posv], fneg1, mask=m0)

                def rbody(jj, carry):
                    acc, accj, lastv = carry
                    cs = csv[pl.ds(jj * 16, 16)]
                    lastv = jnp.where(cs > 0.0, jj, lastv)
                    acc, accj = fold_acc(jj, cs, acc, accj)
                    return acc, accj, lastv
                acc, accj, lastv = lax.fori_loop(0, nb, rbody,
                                                 (fneg1, izero, ineg1))
                return acc, accj, kept, jnp.max(lastv) + 1

            def keep_fn(args):
                acc, accj, kept, nb = args
                kiv = jnp.full((16,), kept, jnp.int32)
                plsc.store_scatter(kx1, [kiv], bx1, mask=m0)
                plsc.store_scatter(ky1, [kiv], by1, mask=m0)
                plsc.store_scatter(kx2, [kiv], bx2, mask=m0)
                plsc.store_scatter(ky2, [kiv], by2, mask=m0)
                plsc.store_scatter(kar, [kiv], bhc, mask=m0)
                plsc.store_scatter(rsv, [kiv], mv, mask=m0)
                plsc.store_scatter(riv, [kiv], bidxv, mask=m0)
                plsc.store_scatter(csv, [posv], fneg1, mask=m0)

                def sbody(jj, carry):
                    acc, accj, lastv = carry
                    slj = pl.ds(jj * 16, 16)
                    cs = csv[slj]
                    xx1 = jnp.maximum(cx1[slj], bx1)
                    yy1 = jnp.maximum(cy1[slj], by1)
                    xx2 = jnp.minimum(cx2[slj], bx2)
                    yy2 = jnp.minimum(cy2[slj], by2)
                    inter = (jnp.maximum(xx2 - xx1, 0.0)
                             * jnp.maximum(yy2 - yy1, 0.0))
                    bad = inter * (1.0 + _IOU_THR) > chc[slj] + bah
                    ncs = jnp.where(bad, -1.0, cs)
                    csv[slj] = ncs
                    lastv = jnp.where(ncs > 0.0, jj, lastv)
                    acc, accj = fold_acc(jj, ncs, acc, accj)
                    return acc, accj, lastv
                acc, accj, lastv = lax.fori_loop(0, nb, sbody,
                                                 (fneg1, izero, ineg1))
                return acc, accj, kept + 1, jnp.max(lastv) + 1

            acc, accj, kept, nb = lax.cond(sup, sup_fn, keep_fn,
                                           (acc, accj, kept, nb))
            return (kept, kp_prev, remaining, leftover, tcur, acc, accj, nb)

        def wcond(st):
            kept, kp_prev, remaining, leftover, tcur, acc, accj, nb = st
            return (kept < _MAX_DET) & ((jnp.max(acc) > 0.0)
                                        | (remaining > 0))

        def wbody(st):
            need_refill = jnp.max(st[5]) <= 0.0
            return lax.cond(need_refill, refill_branch, pick_branch, st)

        st0 = (jnp.int32(0), jnp.int32(0), remaining0, jnp.int32(0),
               smax, fneg1, izero, jnp.int32(0))
        lax.while_loop(wcond, wbody, st0)

        pltpu.sync_copy(rsv, rec_s_hbm.at[c])
        pltpu.sync_copy(riv, rec_i_hbm.at[c])

    for k in range(3):
        c = wid + _NW * k

        @pl.when(c < _C)
        def _():
            process_class(c)


def _merge_body(rows_ref, rec_s_ref, rec_i_ref,
                out_s_ref, out_l_ref, out_b_ref, ms_ref):
    ms_ref[...] = rec_s_ref[...]
    out_s_ref[...] = jnp.full((1, _RP), -1.0, jnp.float32)
    out_l_ref[...] = jnp.full((1, _RP), -1, jnp.int32)
    out_b_ref[...] = jnp.full((4, _RP), -1.0, jnp.float32)

    fi = (jax.lax.broadcasted_iota(jnp.int32, (_C, _RP), 0) * _RP
          + jax.lax.broadcasted_iota(jnp.int32, (_C, _RP), 1))
    ocol = jax.lax.broadcasted_iota(jnp.int32, (1, _RP), 1)
    brow = jax.lax.broadcasted_iota(jnp.int32, (4, _NP), 1)

    def mbody(j, carry):
        rec = ms_ref[...]
        m2 = jnp.max(rec)
        fidx = jnp.min(jnp.where(rec == m2, fi, _BIG_I))
        kidx = jnp.max(jnp.where(fi == fidx, rec_i_ref[...], -2))
        label = fidx // _RP
        valid = kidx >= 0
        ms_ref[...] = jnp.where(fi == fidx, -2.0, rec)

        coords = jnp.max(jnp.where(brow == kidx, rows_ref[...], -1e9),
                         axis=1, keepdims=True)
        hit = ocol == j
        out_s_ref[...] = jnp.where(hit, jnp.where(valid, m2, -1.0),
                                   out_s_ref[...])
        out_l_ref[...] = jnp.where(hit, jnp.where(valid, label, -1),
                                   out_l_ref[...])
        out_b_ref[...] = jnp.where(hit, jnp.where(valid, coords, -1.0),
                                   out_b_ref[...])
        return carry

    jax.lax.fori_loop(0, _MAX_DET, mbody, 0, unroll=False)


@jax.jit
def kernel(boxes, classes):
    rows = jnp.zeros((4, _NP), jnp.float32).at[:, :_N].set(boxes.T)
    scores_t = jnp.full((_C, _NP), -1.0, jnp.float32).at[:, :_N].set(
        classes.T)

    mesh = plsc.VectorSubcoreMesh(core_axis_name="c", subcore_axis_name="s")
    rec_s, rec_i = pl.kernel(
        _sc_nms,
        out_type=[
            jax.ShapeDtypeStruct((_C, _RP), jnp.float32),
            jax.ShapeDtypeStruct((_C, _RP), jnp.int32),
        ],
        mesh=mesh,
        compiler_params=pltpu.CompilerParams(needs_layout_passes=False),
        scratch_types=[
            pltpu.VMEM((_NP,), jnp.float32),
            pltpu.VMEM((_NP,), jnp.float32),
            pltpu.VMEM((_NP,), jnp.float32),
            pltpu.VMEM((_NP,), jnp.float32),
            pltpu.VMEM((_NP,), jnp.float32),
            pltpu.VMEM((_BUF,), jnp.float32),
            pltpu.VMEM((_BUF,), jnp.int32),
            pltpu.VMEM((_BUF,), jnp.float32),
            pltpu.VMEM((_BUF,), jnp.float32),
            pltpu.VMEM((_BUF,), jnp.float32),
            pltpu.VMEM((_BUF,), jnp.float32),
            pltpu.VMEM((_BUF,), jnp.float32),
            pltpu.VMEM((_BUF,), jnp.float32),
            pltpu.VMEM((_RP,), jnp.float32),
            pltpu.VMEM((_RP,), jnp.float32),
            pltpu.VMEM((_RP,), jnp.float32),
            pltpu.VMEM((_RP,), jnp.float32),
            pltpu.VMEM((_RP,), jnp.float32),
            pltpu.VMEM((_RP,), jnp.float32),
            pltpu.VMEM((_RP,), jnp.int32),
        ],
    )(scores_t, rows)

    out_s, out_l, out_b = pl.pallas_call(
        _merge_body,
        out_shape=[
            jax.ShapeDtypeStruct((1, _RP), jnp.float32),
            jax.ShapeDtypeStruct((1, _RP), jnp.int32),
            jax.ShapeDtypeStruct((4, _RP), jnp.float32),
        ],
        in_specs=[
            pl.BlockSpec(memory_space=pltpu.VMEM),
            pl.BlockSpec(memory_space=pltpu.VMEM),
            pl.BlockSpec(memory_space=pltpu.VMEM),
        ],
        out_specs=[
            pl.BlockSpec(memory_space=pltpu.VMEM),
            pl.BlockSpec(memory_space=pltpu.VMEM),
            pl.BlockSpec(memory_space=pltpu.VMEM),
        ],
        scratch_shapes=[
            pltpu.VMEM((_C, _RP), jnp.float32),
        ],
    )(rows, rec_s, rec_i)

    boxes_out = out_b.T[:_MAX_DET]
    scores_out = out_s[0, :_MAX_DET]
    labels_out = out_l[0, :_MAX_DET]
    return boxes_out, scores_out, labels_out


# speedup vs baseline: 9.7276x; 1.0062x over previous
import jax
import jax.numpy as jnp
from jax import lax
from jax.experimental import pallas as pl
from jax.experimental.pallas import tpu as pltpu
from jax.experimental.pallas import tpu_sc as plsc

_N = 20000
_C = 80
_MAX_DET = 300
_IOU_THR = 0.5
_SCORE_THR = 0.05
_NP = 20096
_RP = 304
_BUF = 256
_NB = _NP // 16
_KB = _RP // 16
_CB = _BUF // 16
_NW = 32
_BIG_I = 2**30


def _io16():
    return lax.broadcasted_iota(jnp.int32, (16,), 0)


def _vmaxsplat(v):
    return plsc.cummax(lax.rev(plsc.cummax(v), (0,)))


def _sc_nms(scores_hbm, rows_hbm, rec_s_hbm, rec_i_hbm,
            x1v, y1v, x2v, y2v, sv, csv, civ,
            cx1, cy1, cx2, cy2, car, chc,
            kx1, ky1, kx2, ky2, kar, rsv, riv):
    wid = lax.axis_index("s") * 2 + lax.axis_index("c")
    io16 = _io16()
    fneg1 = jnp.full((16,), -1.0, jnp.float32)
    izero = jnp.zeros((16,), jnp.int32)

    pltpu.sync_copy(rows_hbm.at[0], x1v)
    pltpu.sync_copy(rows_hbm.at[1], y1v)
    pltpu.sync_copy(rows_hbm.at[2], x2v)
    pltpu.sync_copy(rows_hbm.at[3], y2v)

    def count3(t1, t2, t3):
        def body(i, carry):
            c1, c2, c3 = carry
            v = sv[pl.ds(i * 16, 16)]
            c1 = c1 + jnp.where(v > t1, 1, 0)
            c2 = c2 + jnp.where(v > t2, 1, 0)
            c3 = c3 + jnp.where(v > t3, 1, 0)
            return c1, c2, c3
        c1, c2, c3 = lax.fori_loop(0, _NB, body, (izero, izero, izero))
        return jnp.sum(c1), jnp.sum(c2), jnp.sum(c3)

    def choose_t(tcur):
        tlo0 = jnp.full((16,), _SCORE_THR, jnp.float32)

        def bcond(st):
            found, lo, hi, t, it = st
            return (found == 0) & (it < 24)

        def bbody(st):
            found, lo, hi, t, it = st
            span = hi - lo
            q1 = lo + span * 0.25
            q2 = lo + span * 0.5
            q3 = lo + span * 0.75
            collapsed = jnp.any(q1 <= lo) | jnp.any(q3 >= hi)
            f1, f2, f3 = count3(q1, q2, q3)
            le1 = f1 <= _BUF
            le2 = f2 <= _BUF
            le3 = f3 <= _BUF
            first_le_t = jnp.where(le1, q1, jnp.where(le2, q2, q3))
            f_first = jnp.where(le1, f1, jnp.where(le2, f2, f3))
            accept = le3 & (f_first >= 1)
            new_lo = jnp.where(~le3, q3,
                               jnp.where(~le2, q2, jnp.where(~le1, q1, lo)))
            new_hi = jnp.where(le3, first_le_t, hi)
            nfound = jnp.where(accept, 1, jnp.where(collapsed, 2, 0))
            nt = jnp.where(accept, first_le_t, new_lo)
            return nfound, new_lo, new_hi, nt, it + 1

        found, lo, hi, t, it = lax.while_loop(
            bcond, bbody, (jnp.int32(0), tlo0, tcur, tcur, jnp.int32(0)))
        return jnp.where(found == 1, t, lo)

    def process_class(c):
        pltpu.sync_copy(scores_hbm.at[c], sv)

        def init_body(i, carry):
            cnt, vmax = carry
            sl = pl.ds(i * 16, 16)
            v = sv[sl]
            m = v > _SCORE_THR
            v2 = jnp.where(m, v, -1.0)
            sv[sl] = v2
            return cnt + jnp.where(m, 1, 0), jnp.maximum(vmax, v2)
        cnt16, vmax16 = lax.fori_loop(0, _NB, init_body, (izero, fneg1))
        remaining0 = jnp.sum(cnt16)
        smax = _vmaxsplat(vmax16)

        for j in range(_KB):
            sl = pl.ds(j * 16, 16)
            kx1[sl] = jnp.full((16,), 1e30, jnp.float32)
            ky1[sl] = jnp.full((16,), 1e30, jnp.float32)
            kx2[sl] = jnp.full((16,), 1e30, jnp.float32)
            ky2[sl] = jnp.full((16,), 1e30, jnp.float32)
            kar[sl] = jnp.zeros((16,), jnp.float32)
            rsv[sl] = fneg1
            riv[sl] = jnp.full((16,), -1, jnp.int32)

        def fold_acc(j, cs, acc, accj):
            m = cs > acc
            return jnp.where(m, cs, acc), jnp.where(m, j, accj)

        def refill_branch(st):
            kept, kp_prev, remaining, leftover, tcur, acc, accj, nb = st
            t = lax.cond(
                leftover > 0,
                lambda _: tcur,
                lambda _: lax.cond(
                    remaining <= _BUF,
                    lambda __: jnp.full((16,), _SCORE_THR, jnp.float32),
                    lambda __: choose_t(tcur), 0),
                0)

            for j in range(_CB):
                csv[pl.ds(j * 16, 16)] = fneg1
                civ[pl.ds(j * 16, 16)] = izero

            def cbody(i, carry):
                stored, matches = carry
                sl = pl.ds(i * 16, 16)
                v = sv[sl]
                m = v > t
                mi = jnp.where(m, 1, 0)
                dest = stored + plsc.cumsum(mi) - 1
                okm = m & (dest < _BUF)
                destc = jnp.minimum(jnp.maximum(dest, 0), _BUF - 1)
                plsc.store_scatter(csv, [destc], v, mask=okm)
                plsc.store_scatter(civ, [destc], i * 16 + io16, mask=okm)
                sv[sl] = jnp.where(okm, -1.0, v)
                return (stored + jnp.sum(jnp.where(okm, 1, 0)),
                        matches + jnp.sum(mi))
            stored, matches = lax.fori_loop(0, _NB, cbody,
                                            (jnp.int32(0), jnp.int32(0)))

            nb2 = (stored + 15) // 16

            def gbody(i, carry):
                acc, accj = carry
                sl = pl.ds(i * 16, 16)
                idx = civ[sl]
                a = plsc.load_gather(x1v, [idx])
                b = plsc.load_gather(y1v, [idx])
                d = plsc.load_gather(x2v, [idx])
                e = plsc.load_gather(y2v, [idx])
                cx1[sl] = a
                cy1[sl] = b
                cx2[sl] = d
                cy2[sl] = e
                ar = jnp.maximum(d - a, 0.0) * jnp.maximum(e - b, 0.0)
                car[sl] = ar
                chc[sl] = _IOU_THR * (ar + 1e-8)
                return fold_acc(i, csv[sl], acc, accj)
            acc, accj = lax.fori_loop(0, nb2, gbody, (fneg1, izero))

            bail = stored == 0
            remaining2 = jnp.where(bail, 0, remaining - stored)
            leftover2 = jnp.where(bail, 0, matches - stored)
            return (kept, kept, remaining2, leftover2, t, acc, accj, nb2)

        def pick_branch(st):
            kept, kp_prev, remaining, leftover, tcur, acc, accj, nb = st
            mvs = jnp.max(acc)
            pos = jnp.min(jnp.where(acc >= mvs,
                                    accj * 16 + io16, _BIG_I))
            posv = jnp.full((16,), pos, jnp.int32)
            mv = jnp.full((16,), mvs, jnp.float32)
            bx1 = plsc.load_gather(cx1, [posv])
            by1 = plsc.load_gather(cy1, [posv])
            bx2 = plsc.load_gather(cx2, [posv])
            by2 = plsc.load_gather(cy2, [posv])
            bhc = plsc.load_gather(chc, [posv])
            bah = _IOU_THR * plsc.load_gather(car, [posv])
            bidxv = plsc.load_gather(civ, [posv])

            nkv = (kp_prev + 15) // 16

            def kbody(k, sacc):
                slk = pl.ds(k * 16, 16)
                xx1 = jnp.maximum(kx1[slk], bx1)
                yy1 = jnp.maximum(ky1[slk], by1)
                xx2 = jnp.minimum(kx2[slk], bx2)
                yy2 = jnp.minimum(ky2[slk], by2)
                inter = (jnp.maximum(xx2 - xx1, 0.0)
                         * jnp.maximum(yy2 - yy1, 0.0))
                return sacc | (inter * (1.0 + _IOU_THR) > kar[slk] + bah)
            accv = lax.fori_loop(0, nkv, kbody, jnp.zeros((16,), jnp.bool_))
            sup = jnp.max(jnp.where(accv, 1, 0)) > 0

            m0 = io16 == 0
            ineg1 = jnp.full((16,), -1, jnp.int32)

            def sup_fn(args):
                acc, accj, kept, nb = args
                plsc.store_scatter(csv, [posv], fneg1, mask=m0)

                def rbody(jj, carry):
                    acc, accj, lastv = carry
                    cs = csv[pl.ds(jj * 16, 16)]
                    lastv = jnp.where(cs > 0.0, jj, lastv)
                    acc, accj = fold_acc(jj, cs, acc, accj)
                    return acc, accj, lastv
                acc, accj, lastv = lax.fori_loop(0, nb, rbody,
                                                 (fneg1, izero, ineg1))
                return acc, accj, kept, jnp.max(lastv) + 1

            def keep_fn(args):
                acc, accj, kept, nb = args
                kiv = jnp.full((16,), kept, jnp.int32)
                plsc.store_scatter(kx1, [kiv], bx1, mask=m0)
                plsc.store_scatter(ky1, [kiv], by1, mask=m0)
                plsc.store_scatter(kx2, [kiv], bx2, mask=m0)
                plsc.store_scatter(ky2, [kiv], by2, mask=m0)
                plsc.store_scatter(kar, [kiv], bhc, mask=m0)
                plsc.store_scatter(rsv, [kiv], mv, mask=m0)
                plsc.store_scatter(riv, [kiv], bidxv, mask=m0)
                plsc.store_scatter(csv, [posv], fneg1, mask=m0)

                def sbody(jj, carry):
                    acc, accj, lastv = carry
                    slj = pl.ds(jj * 16, 16)
                    cs = csv[slj]
                    xx1 = jnp.maximum(cx1[slj], bx1)
                    yy1 = jnp.maximum(cy1[slj], by1)
                    xx2 = jnp.minimum(cx2[slj], bx2)
                    yy2 = jnp.minimum(cy2[slj], by2)
                    inter = (jnp.maximum(xx2 - xx1, 0.0)
                             * jnp.maximum(yy2 - yy1, 0.0))
                    bad = inter * (1.0 + _IOU_THR) > chc[slj] + bah
                    ncs = jnp.where(bad, -1.0, cs)
                    csv[slj] = ncs
                    lastv = jnp.where(ncs > 0.0, jj, lastv)
                    acc, accj = fold_acc(jj, ncs, acc, accj)
                    return acc, accj, lastv
                acc, accj, lastv = lax.fori_loop(0, nb, sbody,
                                                 (fneg1, izero, ineg1))
                return acc, accj, kept + 1, jnp.max(lastv) + 1

            acc, accj, kept, nb = lax.cond(sup, sup_fn, keep_fn,
                                           (acc, accj, kept, nb))
            return (kept, kp_prev, remaining, leftover, tcur, acc, accj, nb)

        def wcond(st):
            kept, kp_prev, remaining, leftover, tcur, acc, accj, nb = st
            return (kept < _MAX_DET) & ((jnp.max(acc) > 0.0)
                                        | (remaining > 0))

        def wbody(st):
            need_refill = jnp.max(st[5]) <= 0.0
            return lax.cond(need_refill, refill_branch, pick_branch, st)

        st0 = (jnp.int32(0), jnp.int32(0), remaining0, jnp.int32(0),
               smax, fneg1, izero, jnp.int32(0))
        lax.while_loop(wcond, wbody, st0)

        pltpu.sync_copy(rsv, rec_s_hbm.at[c])
        pltpu.sync_copy(riv, rec_i_hbm.at[c])

    for k in range(3):
        c = wid + _NW * k

        @pl.when(c < _C)
        def _():
            process_class(c)


def _merge_body(rows_ref, rec_s_ref, rec_i_ref,
                out_s_ref, out_l_ref, out_b_ref, ms_ref):
    ms_ref[...] = rec_s_ref[...]
    out_s_ref[...] = jnp.full((1, _RP), -1.0, jnp.float32)
    out_l_ref[...] = jnp.full((1, _RP), -1, jnp.int32)
    out_b_ref[...] = jnp.full((4, _RP), -1.0, jnp.float32)

    fi = (jax.lax.broadcasted_iota(jnp.int32, (_C, _RP), 0) * _RP
          + jax.lax.broadcasted_iota(jnp.int32, (_C, _RP), 1))
    ocol = jax.lax.broadcasted_iota(jnp.int32, (1, _RP), 1)
    brow = jax.lax.broadcasted_iota(jnp.int32, (4, _NP), 1)

    def mbody(j, carry):
        rec = ms_ref[...]
        m2 = jnp.max(rec)
        fidx = jnp.min(jnp.where(rec == m2, fi, _BIG_I))
        kidx = jnp.max(jnp.where(fi == fidx, rec_i_ref[...], -2))
        label = fidx // _RP
        valid = kidx >= 0
        ms_ref[...] = jnp.where(fi == fidx, -2.0, rec)

        coords = jnp.max(jnp.where(brow == kidx, rows_ref[...], -1e9),
                         axis=1, keepdims=True)
        hit = ocol == j
        out_s_ref[...] = jnp.where(hit, jnp.where(valid, m2, -1.0),
                                   out_s_ref[...])
        out_l_ref[...] = jnp.where(hit, jnp.where(valid, label, -1),
                                   out_l_ref[...])
        out_b_ref[...] = jnp.where(hit, jnp.where(valid, coords, -1.0),
                                   out_b_ref[...])
        return carry

    jax.lax.fori_loop(0, _MAX_DET, mbody, 0, unroll=False)


@jax.jit
def kernel(boxes, classes):
    rows = jnp.zeros((4, _NP), jnp.float32).at[:, :_N].set(boxes.T)
    scores_t = jnp.full((_C, _NP), -1.0, jnp.float32).at[:, :_N].set(
        classes.T)

    mesh = plsc.VectorSubcoreMesh(core_axis_name="c", subcore_axis_name="s")
    rec_s, rec_i = pl.kernel(
        _sc_nms,
        out_type=[
            jax.ShapeDtypeStruct((_C, _RP), jnp.float32),
            jax.ShapeDtypeStruct((_C, _RP), jnp.int32),
        ],
        mesh=mesh,
        compiler_params=pltpu.CompilerParams(needs_layout_passes=False),
        scratch_types=[
            pltpu.VMEM((_NP,), jnp.float32),
            pltpu.VMEM((_NP,), jnp.float32),
            pltpu.VMEM((_NP,), jnp.float32),
            pltpu.VMEM((_NP,), jnp.float32),
            pltpu.VMEM((_NP,), jnp.float32),
            pltpu.VMEM((_BUF,), jnp.float32),
            pltpu.VMEM((_BUF,), jnp.int32),
            pltpu.VMEM((_BUF,), jnp.float32),
            pltpu.VMEM((_BUF,), jnp.float32),
            pltpu.VMEM((_BUF,), jnp.float32),
            pltpu.VMEM((_BUF,), jnp.float32),
            pltpu.VMEM((_BUF,), jnp.float32),
            pltpu.VMEM((_BUF,), jnp.float32),
            pltpu.VMEM((_RP,), jnp.float32),
            pltpu.VMEM((_RP,), jnp.float32),
            pltpu.VMEM((_RP,), jnp.float32),
            pltpu.VMEM((_RP,), jnp.float32),
            pltpu.VMEM((_RP,), jnp.float32),
            pltpu.VMEM((_RP,), jnp.float32),
            pltpu.VMEM((_RP,), jnp.int32),
        ],
    )(scores_t, rows)

    out_s, out_l, out_b = pl.pallas_call(
        _merge_body,
        out_shape=[
            jax.ShapeDtypeStruct((1, _RP), jnp.float32),
            jax.ShapeDtypeStruct((1, _RP), jnp.int32),
            jax.ShapeDtypeStruct((4, _RP), jnp.float32),
        ],
        in_specs=[
            pl.BlockSpec(memory_space=pltpu.VMEM),
            pl.BlockSpec(memory_space=pltpu.VMEM),
            pl.BlockSpec(memory_space=pltpu.VMEM),
        ],
        out_specs=[
            pl.BlockSpec(memory_space=pltpu.VMEM),
            pl.BlockSpec(memory_space=pltpu.VMEM),
            pl.BlockSpec(memory_space=pltpu.VMEM),
        ],
        scratch_shapes=[
            pltpu.VMEM((_C, _RP), jnp.float32),
        ],
    )(rows, rec_s, rec_i)

    boxes_out = out_b.T[:_MAX_DET]
    scores_out = out_s[0, :_MAX_DET]
    labels_out = out_l[0, :_MAX_DET]
    return boxes_out, scores_out, labels_out


# speedup vs baseline: 9.7769x; 1.0051x over previous
import jax
import jax.numpy as jnp
from jax import lax
from jax.experimental import pallas as pl
from jax.experimental.pallas import tpu as pltpu
from jax.experimental.pallas import tpu_sc as plsc

_N = 20000
_C = 80
_MAX_DET = 300
_IOU_THR = 0.5
_SCORE_THR = 0.05
_NP = 20096
_RP = 304
_BUF = 256
_NB = _NP // 16
_KB = _RP // 16
_CB = _BUF // 16
_NW = 32
_BIG_I = 2**30


def _io16():
    return lax.broadcasted_iota(jnp.int32, (16,), 0)


def _vmaxsplat(v):
    return plsc.cummax(lax.rev(plsc.cummax(v), (0,)))


def _sc_nms(scores_hbm, rows_hbm, rec_s_hbm, rec_i_hbm,
            x1v, y1v, x2v, y2v, sv, csv, civ,
            cx1, cy1, cx2, cy2, car, chc,
            kx1, ky1, kx2, ky2, kar, rsv, riv):
    wid = lax.axis_index("s") * 2 + lax.axis_index("c")
    io16 = _io16()
    fneg1 = jnp.full((16,), -1.0, jnp.float32)
    izero = jnp.zeros((16,), jnp.int32)

    pltpu.sync_copy(rows_hbm.at[0], x1v)
    pltpu.sync_copy(rows_hbm.at[1], y1v)
    pltpu.sync_copy(rows_hbm.at[2], x2v)
    pltpu.sync_copy(rows_hbm.at[3], y2v)

    def count3(t1, t2, t3):
        def body(i, carry):
            c1, c2, c3 = carry
            v = sv[pl.ds(i * 16, 16)]
            c1 = c1 + jnp.where(v > t1, 1, 0)
            c2 = c2 + jnp.where(v > t2, 1, 0)
            c3 = c3 + jnp.where(v > t3, 1, 0)
            return c1, c2, c3
        c1, c2, c3 = lax.fori_loop(0, _NB, body, (izero, izero, izero))
        return jnp.sum(c1), jnp.sum(c2), jnp.sum(c3)

    def choose_t(tcur):
        tlo0 = jnp.full((16,), _SCORE_THR, jnp.float32)

        def bcond(st):
            found, lo, hi, t, it = st
            return (found == 0) & (it < 24)

        def bbody(st):
            found, lo, hi, t, it = st
            span = hi - lo
            q1 = lo + span * 0.25
            q2 = lo + span * 0.5
            q3 = lo + span * 0.75
            collapsed = jnp.any(q1 <= lo) | jnp.any(q3 >= hi)
            f1, f2, f3 = count3(q1, q2, q3)
            le1 = f1 <= _BUF
            le2 = f2 <= _BUF
            le3 = f3 <= _BUF
            first_le_t = jnp.where(le1, q1, jnp.where(le2, q2, q3))
            f_first = jnp.where(le1, f1, jnp.where(le2, f2, f3))
            accept = le3 & (f_first >= 1)
            new_lo = jnp.where(~le3, q3,
                               jnp.where(~le2, q2, jnp.where(~le1, q1, lo)))
            new_hi = jnp.where(le3, first_le_t, hi)
            nfound = jnp.where(accept, 1, jnp.where(collapsed, 2, 0))
            nt = jnp.where(accept, first_le_t, new_lo)
            return nfound, new_lo, new_hi, nt, it + 1

        found, lo, hi, t, it = lax.while_loop(
            bcond, bbody, (jnp.int32(0), tlo0, tcur, tcur, jnp.int32(0)))
        return jnp.where(found == 1, t, lo)

    def process_class(c):
        pltpu.sync_copy(scores_hbm.at[c], sv)

        def init_body(i, carry):
            cnt, vmax = carry
            v = sv[pl.ds(i * 16, 16)]
            m = v > _SCORE_THR
            return cnt + jnp.where(m, 1, 0), jnp.maximum(vmax, v)
        cnt16, vmax16 = lax.fori_loop(0, _NB, init_body, (izero, fneg1))
        remaining0 = jnp.sum(cnt16)
        smax = _vmaxsplat(vmax16)

        for j in range(_KB):
            sl = pl.ds(j * 16, 16)
            kx1[sl] = jnp.full((16,), 1e30, jnp.float32)
            ky1[sl] = jnp.full((16,), 1e30, jnp.float32)
            kx2[sl] = jnp.full((16,), 1e30, jnp.float32)
            ky2[sl] = jnp.full((16,), 1e30, jnp.float32)
            kar[sl] = jnp.zeros((16,), jnp.float32)
            rsv[sl] = fneg1
            riv[sl] = jnp.full((16,), -1, jnp.int32)

        def fold_acc(j, cs, acc, accj):
            m = cs > acc
            return jnp.where(m, cs, acc), jnp.where(m, j, accj)

        def refill_branch(st):
            kept, kp_prev, remaining, leftover, tcur, acc, accj, nb = st
            t = lax.cond(
                leftover > 0,
                lambda _: tcur,
                lambda _: lax.cond(
                    remaining <= _BUF,
                    lambda __: jnp.full((16,), _SCORE_THR, jnp.float32),
                    lambda __: choose_t(tcur), 0),
                0)

            for j in range(_CB):
                csv[pl.ds(j * 16, 16)] = fneg1
                civ[pl.ds(j * 16, 16)] = izero

            def cbody(i, carry):
                stored, matches = carry
                sl = pl.ds(i * 16, 16)
                v = sv[sl]
                m = v > t
                mi = jnp.where(m, 1, 0)
                dest = stored + plsc.cumsum(mi) - 1
                okm = m & (dest < _BUF)
                destc = jnp.minimum(jnp.maximum(dest, 0), _BUF - 1)
                plsc.store_scatter(csv, [destc], v, mask=okm)
                plsc.store_scatter(civ, [destc], i * 16 + io16, mask=okm)
                sv[sl] = jnp.where(okm, -1.0, v)
                return (stored + jnp.sum(jnp.where(okm, 1, 0)),
                        matches + jnp.sum(mi))
            stored, matches = lax.fori_loop(0, _NB, cbody,
                                            (jnp.int32(0), jnp.int32(0)))

            nb2 = (stored + 15) // 16

            def gbody(i, carry):
                acc, accj = carry
                sl = pl.ds(i * 16, 16)
                idx = civ[sl]
                a = plsc.load_gather(x1v, [idx])
                b = plsc.load_gather(y1v, [idx])
                d = plsc.load_gather(x2v, [idx])
                e = plsc.load_gather(y2v, [idx])
                cx1[sl] = a
                cy1[sl] = b
                cx2[sl] = d
                cy2[sl] = e
                ar = jnp.maximum(d - a, 0.0) * jnp.maximum(e - b, 0.0)
                car[sl] = ar
                chc[sl] = _IOU_THR * (ar + 1e-8)
                return fold_acc(i, csv[sl], acc, accj)
            acc, accj = lax.fori_loop(0, nb2, gbody, (fneg1, izero))

            bail = stored == 0
            remaining2 = jnp.where(bail, 0, remaining - stored)
            leftover2 = jnp.where(bail, 0, matches - stored)
            return (kept, kept, remaining2, leftover2, t, acc, accj, nb2)

        def pick_branch(st):
            kept, kp_prev, remaining, leftover, tcur, acc, accj, nb = st
            mvs = jnp.max(acc)
            pos = jnp.min(jnp.where(acc >= mvs,
                                    accj * 16 + io16, _BIG_I))
            posv = jnp.full((16,), pos, jnp.int32)
            mv = jnp.full((16,), mvs, jnp.float32)
            bx1 = plsc.load_gather(cx1, [posv])
            by1 = plsc.load_gather(cy1, [posv])
            bx2 = plsc.load_gather(cx2, [posv])
            by2 = plsc.load_gather(cy2, [posv])
            bhc = plsc.load_gather(chc, [posv])
            bah = _IOU_THR * plsc.load_gather(car, [posv])
            bidxv = plsc.load_gather(civ, [posv])

            nkv = (kp_prev + 15) // 16

            def kbody(k, sacc):
                slk = pl.ds(k * 16, 16)
                xx1 = jnp.maximum(kx1[slk], bx1)
                yy1 = jnp.maximum(ky1[slk], by1)
                xx2 = jnp.minimum(kx2[slk], bx2)
                yy2 = jnp.minimum(ky2[slk], by2)
                inter = (jnp.maximum(xx2 - xx1, 0.0)
                         * jnp.maximum(yy2 - yy1, 0.0))
                return sacc | (inter * (1.0 + _IOU_THR) > kar[slk] + bah)
            accv = lax.fori_loop(0, nkv, kbody, jnp.zeros((16,), jnp.bool_))
            sup = jnp.max(jnp.where(accv, 1, 0)) > 0

            m0 = io16 == 0
            ineg1 = jnp.full((16,), -1, jnp.int32)

            def sup_fn(args):
                acc, accj, kept, nb = args
                plsc.store_scatter(csv, [posv], fneg1, mask=m0)

                def rbody(jj, carry):
                    acc, accj, lastv = carry
                    cs = csv[pl.ds(jj * 16, 16)]
                    lastv = jnp.where(cs > 0.0, jj, lastv)
                    acc, accj = fold_acc(jj, cs, acc, accj)
                    return acc, accj, lastv
                acc, accj, lastv = lax.fori_loop(0, nb, rbody,
                                                 (fneg1, izero, ineg1))
                return acc, accj, kept, jnp.max(lastv) + 1

            def keep_fn(args):
                acc, accj, kept, nb = args
                kiv = jnp.full((16,), kept, jnp.int32)
                plsc.store_scatter(kx1, [kiv], bx1, mask=m0)
                plsc.store_scatter(ky1, [kiv], by1, mask=m0)
                plsc.store_scatter(kx2, [kiv], bx2, mask=m0)
                plsc.store_scatter(ky2, [kiv], by2, mask=m0)
                plsc.store_scatter(kar, [kiv], bhc, mask=m0)
                plsc.store_scatter(rsv, [kiv], mv, mask=m0)
                plsc.store_scatter(riv, [kiv], bidxv, mask=m0)
                plsc.store_scatter(csv, [posv], fneg1, mask=m0)

                def sbody(jj, carry):
                    acc, accj, lastv = carry
                    slj = pl.ds(jj * 16, 16)
                    cs = csv[slj]
                    xx1 = jnp.maximum(cx1[slj], bx1)
                    yy1 = jnp.maximum(cy1[slj], by1)
                    xx2 = jnp.minimum(cx2[slj], bx2)
                    yy2 = jnp.minimum(cy2[slj], by2)
                    inter = (jnp.maximum(xx2 - xx1, 0.0)
                             * jnp.maximum(yy2 - yy1, 0.0))
                    bad = inter * (1.0 + _IOU_THR) > chc[slj] + bah
                    ncs = jnp.where(bad, -1.0, cs)
                    csv[slj] = ncs
                    lastv = jnp.where(ncs > 0.0, jj, lastv)
                    acc, accj = fold_acc(jj, ncs, acc, accj)
                    return acc, accj, lastv
                acc, accj, lastv = lax.fori_loop(0, nb, sbody,
                                                 (fneg1, izero, ineg1))
                return acc, accj, kept + 1, jnp.max(lastv) + 1

            acc, accj, kept, nb = lax.cond(sup, sup_fn, keep_fn,
                                           (acc, accj, kept, nb))
            return (kept, kp_prev, remaining, leftover, tcur, acc, accj, nb)

        def wcond(st):
            kept, kp_prev, remaining, leftover, tcur, acc, accj, nb = st
            return (kept < _MAX_DET) & ((jnp.max(acc) > 0.0)
                                        | (remaining > 0))

        def wbody(st):
            need_refill = jnp.max(st[5]) <= 0.0
            return lax.cond(need_refill, refill_branch, pick_branch, st)

        st0 = (jnp.int32(0), jnp.int32(0), remaining0, jnp.int32(0),
               smax, fneg1, izero, jnp.int32(0))
        lax.while_loop(wcond, wbody, st0)

        pltpu.sync_copy(rsv, rec_s_hbm.at[c])
        pltpu.sync_copy(riv, rec_i_hbm.at[c])

    for k in range(3):
        c = wid + _NW * k

        @pl.when(c < _C)
        def _():
            process_class(c)


def _merge_body(rows_ref, rec_s_ref, rec_i_ref,
                out_s_ref, out_l_ref, out_b_ref, ms_ref):
    ms_ref[...] = rec_s_ref[...]
    out_s_ref[...] = jnp.full((1, _RP), -1.0, jnp.float32)
    out_l_ref[...] = jnp.full((1, _RP), -1, jnp.int32)
    out_b_ref[...] = jnp.full((4, _RP), -1.0, jnp.float32)

    fi = (jax.lax.broadcasted_iota(jnp.int32, (_C, _RP), 0) * _RP
          + jax.lax.broadcasted_iota(jnp.int32, (_C, _RP), 1))
    ocol = jax.lax.broadcasted_iota(jnp.int32, (1, _RP), 1)
    brow = jax.lax.broadcasted_iota(jnp.int32, (4, _NP), 1)

    def mbody(j, carry):
        rec = ms_ref[...]
        m2 = jnp.max(rec)
        fidx = jnp.min(jnp.where(rec == m2, fi, _BIG_I))
        kidx = jnp.max(jnp.where(fi == fidx, rec_i_ref[...], -2))
        label = fidx // _RP
        valid = kidx >= 0
        ms_ref[...] = jnp.where(fi == fidx, -2.0, rec)

        coords = jnp.max(jnp.where(brow == kidx, rows_ref[...], -1e9),
                         axis=1, keepdims=True)
        hit = ocol == j
        out_s_ref[...] = jnp.where(hit, jnp.where(valid, m2, -1.0),
                                   out_s_ref[...])
        out_l_ref[...] = jnp.where(hit, jnp.where(valid, label, -1),
                                   out_l_ref[...])
        out_b_ref[...] = jnp.where(hit, jnp.where(valid, coords, -1.0),
                                   out_b_ref[...])
        return carry

    jax.lax.fori_loop(0, _MAX_DET, mbody, 0, unroll=False)


@jax.jit
def kernel(boxes, classes):
    rows = jnp.zeros((4, _NP), jnp.float32).at[:, :_N].set(boxes.T)
    scores_t = jnp.full((_C, _NP), -1.0, jnp.float32).at[:, :_N].set(
        classes.T)

    mesh = plsc.VectorSubcoreMesh(core_axis_name="c", subcore_axis_name="s")
    rec_s, rec_i = pl.kernel(
        _sc_nms,
        out_type=[
            jax.ShapeDtypeStruct((_C, _RP), jnp.float32),
            jax.ShapeDtypeStruct((_C, _RP), jnp.int32),
        ],
        mesh=mesh,
        compiler_params=pltpu.CompilerParams(needs_layout_passes=False),
        scratch_types=[
            pltpu.VMEM((_NP,), jnp.float32),
            pltpu.VMEM((_NP,), jnp.float32),
            pltpu.VMEM((_NP,), jnp.float32),
            pltpu.VMEM((_NP,), jnp.float32),
            pltpu.VMEM((_NP,), jnp.float32),
            pltpu.VMEM((_BUF,), jnp.float32),
            pltpu.VMEM((_BUF,), jnp.int32),
            pltpu.VMEM((_BUF,), jnp.float32),
            pltpu.VMEM((_BUF,), jnp.float32),
            pltpu.VMEM((_BUF,), jnp.float32),
            pltpu.VMEM((_BUF,), jnp.float32),
            pltpu.VMEM((_BUF,), jnp.float32),
            pltpu.VMEM((_BUF,), jnp.float32),
            pltpu.VMEM((_RP,), jnp.float32),
            pltpu.VMEM((_RP,), jnp.float32),
            pltpu.VMEM((_RP,), jnp.float32),
            pltpu.VMEM((_RP,), jnp.float32),
            pltpu.VMEM((_RP,), jnp.float32),
            pltpu.VMEM((_RP,), jnp.float32),
            pltpu.VMEM((_RP,), jnp.int32),
        ],
    )(scores_t, rows)

    out_s, out_l, out_b = pl.pallas_call(
        _merge_body,
        out_shape=[
            jax.ShapeDtypeStruct((1, _RP), jnp.float32),
            jax.ShapeDtypeStruct((1, _RP), jnp.int32),
            jax.ShapeDtypeStruct((4, _RP), jnp.float32),
        ],
        in_specs=[
            pl.BlockSpec(memory_space=pltpu.VMEM),
            pl.BlockSpec(memory_space=pltpu.VMEM),
            pl.BlockSpec(memory_space=pltpu.VMEM),
        ],
        out_specs=[
            pl.BlockSpec(memory_space=pltpu.VMEM),
            pl.BlockSpec(memory_space=pltpu.VMEM),
            pl.BlockSpec(memory_space=pltpu.VMEM),
        ],
        scratch_shapes=[
            pltpu.VMEM((_C, _RP), jnp.float32),
        ],
    )(rows, rec_s, rec_i)

    boxes_out = out_b.T[:_MAX_DET]
    scores_out = out_s[0, :_MAX_DET]
    labels_out = out_l[0, :_MAX_DET]
    return boxes_out, scores_out, labels_out
